# Initial kernel scaffold; baseline (speedup 1.0000x reference)
import jax
import jax.numpy as jnp
from jax.experimental import pallas as pl


def kernel(x, edge_index, edge_attr, batch, W1, as1, ad1, We1, ae1, b1, W2, as2, ad2, We2, ae2, b2):
    raise NotImplementedError("write your pallas kernel here")


# speedup vs baseline: 10.2228x; 10.2228x over previous
import jax
import jax.numpy as jnp
from jax import lax
from jax.experimental import pallas as pl
from jax.experimental.pallas import tpu as pltpu
from jax.experimental.pallas import tpu_sc as plsc

N = 10000
E = 320000
D = 128
DE = 16
G = 16

NC = 2
NS = 16
L = 16

N_PAD = 10240
E_PAD = 327680
EPT = E_PAD // NS
K = 128
CPT = EPT // K
NPASS = 2
HCPT = CPT // NPASS
DH = D // NC
ROWS_PER_TILE = N_PAD // NS

NB = 256
EB = 2048


def _edge_dense_body(ea_ref, we_ref, aev_ref, eatt_ref, easum_ref):
    i = pl.program_id(0)
    w_e = we_ref[...] @ aev_ref[...]
    eatt_ref[...] = ea_ref[...] @ w_e

    @pl.when(i == 0)
    def _():
        easum_ref[...] = jnp.zeros_like(easum_ref)

    easum_ref[...] += jnp.sum(ea_ref[...], axis=0, keepdims=True)


def _edge_dense(eap, we, aev):
    grid = E_PAD // EB
    return pl.pallas_call(
        _edge_dense_body,
        grid=(grid,),
        in_specs=[
            pl.BlockSpec((EB, DE), lambda i: (i, 0)),
            pl.BlockSpec((DE, D), lambda i: (0, 0)),
            pl.BlockSpec((D, 1), lambda i: (0, 0)),
        ],
        out_specs=[
            pl.BlockSpec((EB, 1), lambda i: (i, 0)),
            pl.BlockSpec((1, DE), lambda i: (0, 0)),
        ],
        out_shape=[
            jax.ShapeDtypeStruct((E_PAD, 1), jnp.float32),
            jax.ShapeDtypeStruct((1, DE), jnp.float32),
        ],
    )(eap, we, aev)


def _node_dense_body(x_ref, w_ref, as_ref, ad_ref, we_ref, aev_ref, easum_ref,
                     h0_ref, h1_ref, hs_ref, hd_ref, sh_ref):
    h = x_ref[...] @ w_ref[...]
    h0_ref[...] = h[:, :DH]
    h1_ref[...] = h[:, DH:]
    hs = h @ as_ref[...]
    hd = h @ ad_ref[...]
    w_e = we_ref[...] @ aev_ref[...]
    c = (easum_ref[...] @ w_e)[0, 0] * (1.0 / E)
    t = hs + hd + c
    hs_ref[...] = hs
    hd_ref[...] = hd
    sh_ref[...] = jnp.where(t >= 0.0, t, 0.2 * t)


def _node_dense(xp, w, asv, adv, we, aev, easum):
    grid = N_PAD // NB
    return pl.pallas_call(
        _node_dense_body,
        grid=(grid,),
        in_specs=[
            pl.BlockSpec((NB, D), lambda i: (i, 0)),
            pl.BlockSpec((D, D), lambda i: (0, 0)),
            pl.BlockSpec((D, 1), lambda i: (0, 0)),
            pl.BlockSpec((D, 1), lambda i: (0, 0)),
            pl.BlockSpec((DE, D), lambda i: (0, 0)),
            pl.BlockSpec((D, 1), lambda i: (0, 0)),
            pl.BlockSpec((1, DE), lambda i: (0, 0)),
        ],
        out_specs=[
            pl.BlockSpec((NB, DH), lambda i: (i, 0)),
            pl.BlockSpec((NB, DH), lambda i: (i, 0)),
            pl.BlockSpec((NB, 1), lambda i: (i, 0)),
            pl.BlockSpec((NB, 1), lambda i: (i, 0)),
            pl.BlockSpec((NB, 1), lambda i: (i, 0)),
        ],
        out_shape=[
            jax.ShapeDtypeStruct((N_PAD, DH), jnp.float32),
            jax.ShapeDtypeStruct((N_PAD, DH), jnp.float32),
            jax.ShapeDtypeStruct((N_PAD, 1), jnp.float32),
            jax.ShapeDtypeStruct((N_PAD, 1), jnp.float32),
            jax.ShapeDtypeStruct((N_PAD, 1), jnp.float32),
        ],
    )(xp, w, asv, adv, we, aev, easum)


def _sc_layer_body(hs_hbm, hd_hbm, sh_hbm, eatt_hbm, src_hbm, dst_hbm,
                   h0_hbm, h1_hbm, out_hbm, den_hbm,
                   hs_v, hd_v, sh_v, eatt_v, src_v, dst_v, ex_c, rows_v,
                   out_sh, den_sh, gsem):
    c = lax.axis_index("c")
    s = lax.axis_index("s")
    cbase = s * CPT

    pltpu.sync_copy(hs_hbm, hs_v)
    pltpu.sync_copy(hd_hbm, hd_v)
    pltpu.sync_copy(sh_hbm, sh_v)

    def _zrow(i, _):
        for k in range(DH // L):
            rows_v[i, pl.ds(k * L, L)] = jnp.zeros((L,), jnp.float32)
        return 0

    lax.fori_loop(0, K, _zrow, 0)
    rbase = s * ROWS_PER_TILE
    for t in range(ROWS_PER_TILE // K):
        pltpu.sync_copy(rows_v, out_sh.at[pl.ds(rbase + t * K, K)])

    @pl.when(c == 0)
    def _():
        for t in range(ROWS_PER_TILE // DH):
            pltpu.sync_copy(rows_v.at[0],
                            den_sh.at[pl.ds(rbase + t * DH, DH)])

    plsc.subcore_barrier()

    def _run_pass(p):
        pbase = cbase + p * HCPT
        pltpu.sync_copy(eatt_hbm.at[pl.ds(pbase, HCPT)], eatt_v)
        pltpu.sync_copy(src_hbm.at[pl.ds(pbase, HCPT)], src_v)
        pltpu.sync_copy(dst_hbm.at[pl.ds(pbase, HCPT)], dst_v)
        lax.fori_loop(0, HCPT, _chunk, 0)

    def _chunk(j, _):
        for k in range(K // L):
            sl = pl.ds(k * L, L)
            isrc = src_v[j, sl]
            idst = dst_v[j, sl]
            av = plsc.load_gather(hs_v, [isrc])
            bv = plsc.load_gather(hd_v, [idst])
            shv = plsc.load_gather(sh_v, [idst])
            al = av + bv + eatt_v[j, sl]
            al = jnp.where(al >= 0.0, al, 0.2 * al)
            ex_c[sl] = jnp.exp(jnp.minimum(al - shv, 80.0))

        @pl.when(c == 0)
        def _():
            pltpu.sync_copy(ex_c, den_sh.at[dst_v.at[j]], add=True)
            pltpu.async_copy(h0_hbm.at[src_v.at[j]], rows_v, gsem).wait()

        @pl.when(c == 1)
        def _():
            pltpu.async_copy(h1_hbm.at[src_v.at[j]], rows_v, gsem).wait()

        def _scale(b, _):
            exv = ex_c[pl.ds(b * L, L)]
            base = b * L
            for e in range(L):
                sv = exv[e]
                for k in range(DH // L):
                    sl = pl.ds(k * L, L)
                    rows_v[base + e, sl] = rows_v[base + e, sl] * sv
            return 0

        lax.fori_loop(0, K // L, _scale, 0)
        pltpu.sync_copy(rows_v, out_sh.at[dst_v.at[j]], add=True)
        return 0

    for p in range(NPASS):
        _run_pass(p)
    plsc.subcore_barrier()

    pltpu.sync_copy(out_sh.at[pl.ds(rbase, ROWS_PER_TILE)],
                    out_hbm.at[c, pl.ds(rbase, ROWS_PER_TILE)])

    @pl.when(c == 0)
    def _():
        pltpu.sync_copy(den_sh.at[pl.ds(rbase, ROWS_PER_TILE)],
                        den_hbm.at[pl.ds(rbase, ROWS_PER_TILE)])


_sc_layer = pl.kernel(
    _sc_layer_body,
    out_type=[
        jax.ShapeDtypeStruct((NC, N_PAD, DH), jnp.float32),
        jax.ShapeDtypeStruct((N_PAD,), jnp.float32),
    ],
    mesh=plsc.VectorSubcoreMesh(core_axis_name="c", subcore_axis_name="s"),
    compiler_params=pltpu.CompilerParams(
        use_tc_tiling_on_sc=False, needs_layout_passes=False),
    scratch_types=[
        pltpu.VMEM((N_PAD,), jnp.float32),
        pltpu.VMEM((N_PAD,), jnp.float32),
        pltpu.VMEM((N_PAD,), jnp.float32),
        pltpu.VMEM((HCPT, K), jnp.float32),
        pltpu.VMEM((HCPT, K), jnp.int32),
        pltpu.VMEM((HCPT, K), jnp.int32),
        pltpu.VMEM((K,), jnp.float32),
        pltpu.VMEM((K, DH), jnp.float32),
        pltpu.VMEM_SHARED((N_PAD, DH), jnp.float32),
        pltpu.VMEM_SHARED((N_PAD,), jnp.float32),
        pltpu.SemaphoreType.DMA,
    ],
)


def _epilogue_body(p0_ref, p1_ref, h0_ref, h1_ref, d_ref, b_ref, y_ref):
    den = d_ref[...] + 1.0
    msg = jnp.concatenate(
        [p0_ref[...] + h0_ref[...], p1_ref[...] + h1_ref[...]], axis=-1)
    y = msg / den + b_ref[...]
    y_ref[...] = y * (1.0 / (1.0 + jnp.exp(-y)))


def _epilogue(p0, p1, h0, h1, d, b2d):
    grid = N_PAD // NB
    return pl.pallas_call(
        _epilogue_body,
        grid=(grid,),
        in_specs=[
            pl.BlockSpec((NB, DH), lambda i: (i, 0)),
            pl.BlockSpec((NB, DH), lambda i: (i, 0)),
            pl.BlockSpec((NB, DH), lambda i: (i, 0)),
            pl.BlockSpec((NB, DH), lambda i: (i, 0)),
            pl.BlockSpec((NB, 1), lambda i: (i, 0)),
            pl.BlockSpec((1, D), lambda i: (0, 0)),
        ],
        out_specs=pl.BlockSpec((NB, D), lambda i: (i, 0)),
        out_shape=jax.ShapeDtypeStruct((N_PAD, D), jnp.float32),
    )(p0, p1, h0, h1, d, b2d)


def _epilogue_pool_body(p0_ref, p1_ref, h0_ref, h1_ref, d_ref, b_ref, bid_ref,
                        pooled_ref, cnt_ref):
    i = pl.program_id(0)

    @pl.when(i == 0)
    def _():
        pooled_ref[...] = jnp.zeros_like(pooled_ref)
        cnt_ref[...] = jnp.zeros_like(cnt_ref)

    den = d_ref[...] + 1.0
    msg = jnp.concatenate(
        [p0_ref[...] + h0_ref[...], p1_ref[...] + h1_ref[...]], axis=-1)
    y = msg / den + b_ref[...]
    y = y * (1.0 / (1.0 + jnp.exp(-y)))
    bid = bid_ref[...]
    ones = jnp.ones_like(y)
    for g in range(G):
        m = bid == g
        pooled_ref[g:g + 1, :] += jnp.sum(jnp.where(m, y, 0.0), axis=0,
                                          keepdims=True)
        cnt_ref[g:g + 1, :] += jnp.sum(jnp.where(m, ones, 0.0), axis=0,
                                       keepdims=True)

    @pl.when(i == pl.num_programs(0) - 1)
    def _():
        pooled_ref[...] = pooled_ref[...] / jnp.maximum(cnt_ref[...], 1.0)


def _epilogue_pool(p0, p1, h0, h1, d, b2d, bid2d):
    grid = N_PAD // NB
    return pl.pallas_call(
        _epilogue_pool_body,
        grid=(grid,),
        in_specs=[
            pl.BlockSpec((NB, DH), lambda i: (i, 0)),
            pl.BlockSpec((NB, DH), lambda i: (i, 0)),
            pl.BlockSpec((NB, DH), lambda i: (i, 0)),
            pl.BlockSpec((NB, DH), lambda i: (i, 0)),
            pl.BlockSpec((NB, 1), lambda i: (i, 0)),
            pl.BlockSpec((1, D), lambda i: (0, 0)),
            pl.BlockSpec((NB, 1), lambda i: (i, 0)),
        ],
        out_specs=pl.BlockSpec((G, D), lambda i: (0, 0)),
        out_shape=jax.ShapeDtypeStruct((G, D), jnp.float32),
        scratch_shapes=[pltpu.VMEM((G, D), jnp.float32)],
    )(p0, p1, h0, h1, d, b2d, bid2d)


def _gat_layer(xp, src2d, dst2d, eap, w, asv, adv, we, aev):
    eatt, easum = _edge_dense(eap, we, aev.reshape(D, 1))
    h0, h1, hs, hd, sh = _node_dense(xp, w, asv.reshape(D, 1),
                                     adv.reshape(D, 1), we,
                                     aev.reshape(D, 1), easum)
    out_parts, den = _sc_layer(
        hs.reshape(N_PAD), hd.reshape(N_PAD), sh.reshape(N_PAD),
        eatt.reshape(E_PAD // K, K), src2d, dst2d, h0, h1)
    return h0, h1, out_parts, den


def kernel(x, edge_index, edge_attr, batch,
           W1, as1, ad1, We1, ae1, b1, W2, as2, ad2, We2, ae2, b2):
    f32 = jnp.float32
    xp = jnp.zeros((N_PAD, D), f32).at[:N].set(x)
    pad_idx = jnp.full((E_PAD - E,), N_PAD - 1, jnp.int32)
    src2d = jnp.concatenate([edge_index[0], pad_idx]).reshape(E_PAD // K, K)
    dst2d = jnp.concatenate([edge_index[1], pad_idx]).reshape(E_PAD // K, K)
    eap = jnp.zeros((E_PAD, DE), f32).at[:E].set(edge_attr)
    bid2d = jnp.full((N_PAD, 1), G, jnp.int32).at[:N, 0].set(batch)

    h0, h1, parts1, den1 = _gat_layer(xp, src2d, dst2d, eap,
                                      W1, as1, ad1, We1, ae1)
    y1 = _epilogue(parts1[0], parts1[1], h0, h1,
                   den1.reshape(N_PAD, 1), b1.reshape(1, D))
    g0, g1, parts2, den2 = _gat_layer(y1, src2d, dst2d, eap,
                                      W2, as2, ad2, We2, ae2)
    pooled = _epilogue_pool(parts2[0], parts2[1], g0, g1,
                            den2.reshape(N_PAD, 1), b2.reshape(1, D), bid2d)
    return pooled


# speedup vs baseline: 13.7843x; 1.3484x over previous
import jax
import jax.numpy as jnp
from jax import lax
from jax.experimental import pallas as pl
from jax.experimental.pallas import tpu as pltpu
from jax.experimental.pallas import tpu_sc as plsc

N = 10000
E = 320000
D = 128
DE = 16
G = 16

NC = 2
NS = 16
L = 16

N_PAD = 10240
E_PAD = 327680
EPT = E_PAD // NS
K = 128
CPT = EPT // K
NPASS = 2
HCPT = CPT // NPASS
DH = D // NC
ROWS_PER_TILE = N_PAD // NS

NB = 256
EB = 2048


def _edge_dense_body(ea_ref, we_ref, aev_ref, eatt_ref, easum_ref):
    i = pl.program_id(0)
    w_e = we_ref[...] @ aev_ref[...]
    eatt_ref[...] = ea_ref[...] @ w_e

    @pl.when(i == 0)
    def _():
        easum_ref[...] = jnp.zeros_like(easum_ref)

    easum_ref[...] += jnp.sum(ea_ref[...], axis=0, keepdims=True)


def _edge_dense(eap, we, aev):
    grid = E_PAD // EB
    return pl.pallas_call(
        _edge_dense_body,
        grid=(grid,),
        in_specs=[
            pl.BlockSpec((EB, DE), lambda i: (i, 0)),
            pl.BlockSpec((DE, D), lambda i: (0, 0)),
            pl.BlockSpec((D, 1), lambda i: (0, 0)),
        ],
        out_specs=[
            pl.BlockSpec((EB, 1), lambda i: (i, 0)),
            pl.BlockSpec((1, DE), lambda i: (0, 0)),
        ],
        out_shape=[
            jax.ShapeDtypeStruct((E_PAD, 1), jnp.float32),
            jax.ShapeDtypeStruct((1, DE), jnp.float32),
        ],
    )(eap, we, aev)


def _node_dense_body(x_ref, w_ref, as_ref, ad_ref, we_ref, aev_ref, easum_ref,
                     h0_ref, h1_ref, hs_ref, hd_ref, sh_ref):
    h = x_ref[...] @ w_ref[...]
    h0_ref[...] = h[:, :DH]
    h1_ref[...] = h[:, DH:]
    hs = h @ as_ref[...]
    hd = h @ ad_ref[...]
    w_e = we_ref[...] @ aev_ref[...]
    c = (easum_ref[...] @ w_e)[0, 0] * (1.0 / E)
    t = hs + hd + c
    hs_ref[...] = hs
    hd_ref[...] = hd
    sh_ref[...] = jnp.where(t >= 0.0, t, 0.2 * t)


def _node_dense(xp, w, asv, adv, we, aev, easum):
    grid = N_PAD // NB
    return pl.pallas_call(
        _node_dense_body,
        grid=(grid,),
        in_specs=[
            pl.BlockSpec((NB, D), lambda i: (i, 0)),
            pl.BlockSpec((D, D), lambda i: (0, 0)),
            pl.BlockSpec((D, 1), lambda i: (0, 0)),
            pl.BlockSpec((D, 1), lambda i: (0, 0)),
            pl.BlockSpec((DE, D), lambda i: (0, 0)),
            pl.BlockSpec((D, 1), lambda i: (0, 0)),
            pl.BlockSpec((1, DE), lambda i: (0, 0)),
        ],
        out_specs=[
            pl.BlockSpec((NB, DH), lambda i: (i, 0)),
            pl.BlockSpec((NB, DH), lambda i: (i, 0)),
            pl.BlockSpec((NB, 1), lambda i: (i, 0)),
            pl.BlockSpec((NB, 1), lambda i: (i, 0)),
            pl.BlockSpec((NB, 1), lambda i: (i, 0)),
        ],
        out_shape=[
            jax.ShapeDtypeStruct((N_PAD, DH), jnp.float32),
            jax.ShapeDtypeStruct((N_PAD, DH), jnp.float32),
            jax.ShapeDtypeStruct((N_PAD, 1), jnp.float32),
            jax.ShapeDtypeStruct((N_PAD, 1), jnp.float32),
            jax.ShapeDtypeStruct((N_PAD, 1), jnp.float32),
        ],
    )(xp, w, asv, adv, we, aev, easum)


def _sc_layer_body(hs_hbm, hd_hbm, sh_hbm, eatt_hbm, src_hbm, dst_hbm,
                   h0_hbm, h1_hbm, out_hbm, den_hbm,
                   hs_v, hd_v, sh_v, eatt_v, src_v, dst_v,
                   ex0, ex1, rows0, rows1,
                   out_sh, den_sh,
                   gsem0, gsem1, ssem0, ssem1, dsem0, dsem1):
    c = lax.axis_index("c")
    s = lax.axis_index("s")
    cbase = s * CPT
    rbufs = (rows0, rows1)
    exbufs = (ex0, ex1)
    gsems = (gsem0, gsem1)
    ssems = (ssem0, ssem1)
    dsems = (dsem0, dsem1)

    pltpu.sync_copy(hs_hbm, hs_v)
    pltpu.sync_copy(hd_hbm, hd_v)
    pltpu.sync_copy(sh_hbm, sh_v)

    def _zrow(i, _):
        for k in range(DH // L):
            rows0[i, pl.ds(k * L, L)] = jnp.zeros((L,), jnp.float32)
        return 0

    lax.fori_loop(0, K, _zrow, 0)
    rbase = s * ROWS_PER_TILE
    for t in range(ROWS_PER_TILE // K):
        pltpu.sync_copy(rows0, out_sh.at[pl.ds(rbase + t * K, K)])

    @pl.when(c == 0)
    def _():
        for t in range(ROWS_PER_TILE // DH):
            pltpu.sync_copy(rows0.at[0],
                            den_sh.at[pl.ds(rbase + t * DH, DH)])

    plsc.subcore_barrier()

    def _issue_gather(j, buf, sem):
        @pl.when(c == 0)
        def _():
            pltpu.async_copy(h0_hbm.at[src_v.at[j]], buf, sem)

        @pl.when(c == 1)
        def _():
            pltpu.async_copy(h1_hbm.at[src_v.at[j]], buf, sem)

    def _chunk(j, b):
        nb = 1 - b
        buf = rbufs[b]
        exb = exbufs[b]

        @pl.when(j >= 1)
        def _():
            pltpu.make_async_copy(rbufs[nb], out_sh.at[dst_v.at[0]],
                                  ssems[nb]).wait()

        @pl.when(j + 1 < HCPT)
        def _():
            _issue_gather(j + 1, rbufs[nb], gsems[nb])

        @pl.when(jnp.logical_and(c == 0, j >= 2))
        def _():
            pltpu.make_async_copy(exb, den_sh.at[dst_v.at[0]],
                                  dsems[b]).wait()

        for k in range(K // L):
            sl = pl.ds(k * L, L)
            isrc = src_v[j, sl]
            idst = dst_v[j, sl]
            av = plsc.load_gather(hs_v, [isrc])
            bv = plsc.load_gather(hd_v, [idst])
            shv = plsc.load_gather(sh_v, [idst])
            al = av + bv + eatt_v[j, sl]
            al = jnp.where(al >= 0.0, al, 0.2 * al)
            exb[sl] = jnp.exp(jnp.minimum(al - shv, 80.0))

        @pl.when(c == 0)
        def _():
            pltpu.async_copy(exb, den_sh.at[dst_v.at[j]], dsems[b], add=True)

        pltpu.make_async_copy(h0_hbm.at[src_v.at[0]], buf, gsems[b]).wait()

        def _scale(v, _):
            exv = exb[pl.ds(v * L, L)]
            base = v * L
            for e in range(L):
                sv = exv[e]
                for k in range(DH // L):
                    sl = pl.ds(k * L, L)
                    buf[base + e, sl] = buf[base + e, sl] * sv
            return 0

        lax.fori_loop(0, K // L, _scale, 0)
        pltpu.async_copy(buf, out_sh.at[dst_v.at[j]], ssems[b], add=True)

    def _pair(i, _):
        _chunk(i * 2, 0)
        _chunk(i * 2 + 1, 1)
        return 0

    for p in range(NPASS):
        pbase = cbase + p * HCPT
        pltpu.sync_copy(eatt_hbm.at[pl.ds(pbase, HCPT)], eatt_v)
        pltpu.sync_copy(src_hbm.at[pl.ds(pbase, HCPT)], src_v)
        pltpu.sync_copy(dst_hbm.at[pl.ds(pbase, HCPT)], dst_v)
        _issue_gather(0, rows0, gsem0)
        lax.fori_loop(0, HCPT // 2, _pair, 0)
        pltpu.make_async_copy(rows1, out_sh.at[dst_v.at[0]], ssem1).wait()

        @pl.when(c == 0)
        def _():
            pltpu.make_async_copy(ex0, den_sh.at[dst_v.at[0]], dsem0).wait()
            pltpu.make_async_copy(ex1, den_sh.at[dst_v.at[0]], dsem1).wait()

    plsc.subcore_barrier()

    pltpu.sync_copy(out_sh.at[pl.ds(rbase, ROWS_PER_TILE)],
                    out_hbm.at[c, pl.ds(rbase, ROWS_PER_TILE)])

    @pl.when(c == 0)
    def _():
        pltpu.sync_copy(den_sh.at[pl.ds(rbase, ROWS_PER_TILE)],
                        den_hbm.at[pl.ds(rbase, ROWS_PER_TILE)])


_sc_layer = pl.kernel(
    _sc_layer_body,
    out_type=[
        jax.ShapeDtypeStruct((NC, N_PAD, DH), jnp.float32),
        jax.ShapeDtypeStruct((N_PAD,), jnp.float32),
    ],
    mesh=plsc.VectorSubcoreMesh(core_axis_name="c", subcore_axis_name="s"),
    compiler_params=pltpu.CompilerParams(
        use_tc_tiling_on_sc=False, needs_layout_passes=False),
    scratch_types=[
        pltpu.VMEM((N_PAD,), jnp.float32),
        pltpu.VMEM((N_PAD,), jnp.float32),
        pltpu.VMEM((N_PAD,), jnp.float32),
        pltpu.VMEM((HCPT, K), jnp.float32),
        pltpu.VMEM((HCPT, K), jnp.int32),
        pltpu.VMEM((HCPT, K), jnp.int32),
        pltpu.VMEM((K,), jnp.float32),
        pltpu.VMEM((K,), jnp.float32),
        pltpu.VMEM((K, DH), jnp.float32),
        pltpu.VMEM((K, DH), jnp.float32),
        pltpu.VMEM_SHARED((N_PAD, DH), jnp.float32),
        pltpu.VMEM_SHARED((N_PAD,), jnp.float32),
        pltpu.SemaphoreType.DMA,
        pltpu.SemaphoreType.DMA,
        pltpu.SemaphoreType.DMA,
        pltpu.SemaphoreType.DMA,
        pltpu.SemaphoreType.DMA,
        pltpu.SemaphoreType.DMA,
    ],
)


def _epilogue_body(p0_ref, p1_ref, h0_ref, h1_ref, d_ref, b_ref, y_ref):
    den = d_ref[...] + 1.0
    msg = jnp.concatenate(
        [p0_ref[...] + h0_ref[...], p1_ref[...] + h1_ref[...]], axis=-1)
    y = msg / den + b_ref[...]
    y_ref[...] = y * (1.0 / (1.0 + jnp.exp(-y)))


def _epilogue(p0, p1, h0, h1, d, b2d):
    grid = N_PAD // NB
    return pl.pallas_call(
        _epilogue_body,
        grid=(grid,),
        in_specs=[
            pl.BlockSpec((NB, DH), lambda i: (i, 0)),
            pl.BlockSpec((NB, DH), lambda i: (i, 0)),
            pl.BlockSpec((NB, DH), lambda i: (i, 0)),
            pl.BlockSpec((NB, DH), lambda i: (i, 0)),
            pl.BlockSpec((NB, 1), lambda i: (i, 0)),
            pl.BlockSpec((1, D), lambda i: (0, 0)),
        ],
        out_specs=pl.BlockSpec((NB, D), lambda i: (i, 0)),
        out_shape=jax.ShapeDtypeStruct((N_PAD, D), jnp.float32),
    )(p0, p1, h0, h1, d, b2d)


def _epilogue_pool_body(p0_ref, p1_ref, h0_ref, h1_ref, d_ref, b_ref, bid_ref,
                        pooled_ref, cnt_ref):
    i = pl.program_id(0)

    @pl.when(i == 0)
    def _():
        pooled_ref[...] = jnp.zeros_like(pooled_ref)
        cnt_ref[...] = jnp.zeros_like(cnt_ref)

    den = d_ref[...] + 1.0
    msg = jnp.concatenate(
        [p0_ref[...] + h0_ref[...], p1_ref[...] + h1_ref[...]], axis=-1)
    y = msg / den + b_ref[...]
    y = y * (1.0 / (1.0 + jnp.exp(-y)))
    bid = bid_ref[...]
    ones = jnp.ones_like(y)
    for g in range(G):
        m = bid == g
        pooled_ref[g:g + 1, :] += jnp.sum(jnp.where(m, y, 0.0), axis=0,
                                          keepdims=True)
        cnt_ref[g:g + 1, :] += jnp.sum(jnp.where(m, ones, 0.0), axis=0,
                                       keepdims=True)

    @pl.when(i == pl.num_programs(0) - 1)
    def _():
        pooled_ref[...] = pooled_ref[...] / jnp.maximum(cnt_ref[...], 1.0)


def _epilogue_pool(p0, p1, h0, h1, d, b2d, bid2d):
    grid = N_PAD // NB
    return pl.pallas_call(
        _epilogue_pool_body,
        grid=(grid,),
        in_specs=[
            pl.BlockSpec((NB, DH), lambda i: (i, 0)),
            pl.BlockSpec((NB, DH), lambda i: (i, 0)),
            pl.BlockSpec((NB, DH), lambda i: (i, 0)),
            pl.BlockSpec((NB, DH), lambda i: (i, 0)),
            pl.BlockSpec((NB, 1), lambda i: (i, 0)),
            pl.BlockSpec((1, D), lambda i: (0, 0)),
            pl.BlockSpec((NB, 1), lambda i: (i, 0)),
        ],
        out_specs=pl.BlockSpec((G, D), lambda i: (0, 0)),
        out_shape=jax.ShapeDtypeStruct((G, D), jnp.float32),
        scratch_shapes=[pltpu.VMEM((G, D), jnp.float32)],
    )(p0, p1, h0, h1, d, b2d, bid2d)


def _gat_layer(xp, src2d, dst2d, eap, w, asv, adv, we, aev):
    eatt, easum = _edge_dense(eap, we, aev.reshape(D, 1))
    h0, h1, hs, hd, sh = _node_dense(xp, w, asv.reshape(D, 1),
                                     adv.reshape(D, 1), we,
                                     aev.reshape(D, 1), easum)
    out_parts, den = _sc_layer(
        hs.reshape(N_PAD), hd.reshape(N_PAD), sh.reshape(N_PAD),
        eatt.reshape(E_PAD // K, K), src2d, dst2d, h0, h1)
    return h0, h1, out_parts, den


def kernel(x, edge_index, edge_attr, batch,
           W1, as1, ad1, We1, ae1, b1, W2, as2, ad2, We2, ae2, b2):
    f32 = jnp.float32
    xp = jnp.zeros((N_PAD, D), f32).at[:N].set(x)
    pad_idx = jnp.full((E_PAD - E,), N_PAD - 1, jnp.int32)
    src2d = jnp.concatenate([edge_index[0], pad_idx]).reshape(E_PAD // K, K)
    dst2d = jnp.concatenate([edge_index[1], pad_idx]).reshape(E_PAD // K, K)
    eap = jnp.zeros((E_PAD, DE), f32).at[:E].set(edge_attr)
    bid2d = jnp.full((N_PAD, 1), G, jnp.int32).at[:N, 0].set(batch)

    h0, h1, parts1, den1 = _gat_layer(xp, src2d, dst2d, eap,
                                      W1, as1, ad1, We1, ae1)
    y1 = _epilogue(parts1[0], parts1[1], h0, h1,
                   den1.reshape(N_PAD, 1), b1.reshape(1, D))
    g0, g1, parts2, den2 = _gat_layer(y1, src2d, dst2d, eap,
                                      W2, as2, ad2, We2, ae2)
    pooled = _epilogue_pool(parts2[0], parts2[1], g0, g1,
                            den2.reshape(N_PAD, 1), b2.reshape(1, D), bid2d)
    return pooled
